# baseline (speedup 1.0000x reference)
import functools

import jax
import jax.numpy as jnp
from jax import lax
from jax.experimental import pallas as pl
from jax.experimental.pallas import tpu as pltpu
from jax.experimental.pallas import tpu_sc as plsc

_NC = 2
_NS = 16
_NW = _NC * _NS

_BN2 = 1000
_G = 64


def _mm_body(x_ref, w_ref, o_ref):
    o_ref[...] = jnp.dot(x_ref[...], w_ref[...],
                         preferred_element_type=jnp.float32)


def _first_matmul(x256, w1d):
    n2, d2 = x256.shape
    return pl.pallas_call(
        _mm_body,
        grid=(n2 // _BN2,),
        in_specs=[pl.BlockSpec((_BN2, d2), lambda i: (i, 0)),
                  pl.BlockSpec((d2, 128), lambda i: (0, 0))],
        out_specs=pl.BlockSpec((_BN2, 128), lambda i: (i, 0)),
        out_shape=jax.ShapeDtypeStruct((n2, 128), jnp.float32),
    )(x256, w1d)


def _fuse_body(y_ref, p_ref, ba_ref, wbd_ref, bb_ref, wnd_ref, o_ref):
    h = jnp.maximum(y_ref[...] + p_ref[0] + p_ref[1] + ba_ref[...], 0.0)
    t = jnp.dot(h, wbd_ref[...], preferred_element_type=jnp.float32)
    t = jnp.maximum(t + bb_ref[...], 0.0)
    o_ref[...] = jnp.dot(t, wnd_ref[...], preferred_element_type=jnp.float32)


def _fuse(y, p, ba2, wbd, bb2, wnd):
    n2 = y.shape[0]
    return pl.pallas_call(
        _fuse_body,
        grid=(n2 // _BN2,),
        in_specs=[pl.BlockSpec((_BN2, 128), lambda i: (i, 0)),
                  pl.BlockSpec((_NC, _BN2, 128), lambda i: (0, i, 0)),
                  pl.BlockSpec((1, 128), lambda i: (0, 0)),
                  pl.BlockSpec((128, 128), lambda i: (0, 0)),
                  pl.BlockSpec((1, 128), lambda i: (0, 0)),
                  pl.BlockSpec((128, 128), lambda i: (0, 0))],
        out_specs=pl.BlockSpec((_BN2, 128), lambda i: (i, 0)),
        out_shape=jax.ShapeDtypeStruct((n2, 128), jnp.float32),
    )(y, p, ba2, wbd, bb2, wnd)


def _final_body(y_ref, p_ref, ba_ref, wbd_ref, bb_ref, b_ref, wl_ref,
                bl_ref, o_ref, acc_ref):
    i = pl.program_id(0)

    @pl.when(i == 0)
    def _():
        acc_ref[...] = jnp.zeros_like(acc_ref)

    h = jnp.maximum(y_ref[...] + p_ref[0] + p_ref[1] + ba_ref[...], 0.0)
    z = jnp.dot(h, wbd_ref[...], preferred_element_type=jnp.float32)
    z = jnp.maximum(z + bb_ref[...], 0.0)
    ids_e = b_ref[0, 0]
    ids_o = b_ref[1, 0]
    seg = lax.broadcasted_iota(jnp.int32, (_G, _BN2), 0)
    oh_e = (seg == ids_e).astype(jnp.float32)
    oh_o = (seg == ids_o).astype(jnp.float32)
    acc_ref[...] += (
        jnp.dot(oh_e, z[:, :64], preferred_element_type=jnp.float32)
        + jnp.dot(oh_o, z[:, 64:], preferred_element_type=jnp.float32))

    @pl.when(i == pl.num_programs(0) - 1)
    def _():
        o_ref[...] = jnp.dot(acc_ref[...], wl_ref[...],
                             preferred_element_type=jnp.float32) + bl_ref[...]


def _final(y, p, ba2, wbd, bb2, batch4, wl, bl):
    n2 = y.shape[0]
    h = wl.shape[0]
    dout = wl.shape[1]
    return pl.pallas_call(
        _final_body,
        grid=(n2 // _BN2,),
        in_specs=[pl.BlockSpec((_BN2, 128), lambda i: (i, 0)),
                  pl.BlockSpec((_NC, _BN2, 128), lambda i: (0, i, 0)),
                  pl.BlockSpec((1, 128), lambda i: (0, 0)),
                  pl.BlockSpec((128, 128), lambda i: (0, 0)),
                  pl.BlockSpec((1, 128), lambda i: (0, 0)),
                  pl.BlockSpec((2, 1, 1, _BN2), lambda i: (0, i, 0, 0)),
                  pl.BlockSpec((h, dout), lambda i: (0, 0)),
                  pl.BlockSpec((1, dout), lambda i: (0, 0))],
        out_specs=pl.BlockSpec((_G, dout), lambda i: (0, 0)),
        out_shape=jax.ShapeDtypeStruct((_G, dout), jnp.float32),
        scratch_shapes=[pltpu.VMEM((_G, h), jnp.float32)],
    )(y, p, ba2, wbd, bb2, batch4, wl, bl)


_NB = 6
_AHEAD = 5


def _make_aggregate(n, h, c_total, k):
    rows_per_tile = n // _NS
    nfull = c_total // _NW
    nextra = c_total - nfull * _NW
    zfull = rows_per_tile // k
    zrem = rows_per_tile - zfull * k
    mesh = plsc.VectorSubcoreMesh(core_axis_name="c", subcore_axis_name="s")
    assert nfull % _NB == 0 and nextra <= _NW

    @functools.partial(
        pl.kernel,
        out_type=jax.ShapeDtypeStruct((_NC, n, h), jnp.float32),
        mesh=mesh,
        compiler_params=pltpu.CompilerParams(use_tc_tiling_on_sc=False),
        scratch_types=[
            pltpu.VMEM((nfull, k), jnp.int32),
            pltpu.VMEM((nfull, k), jnp.int32),
            pltpu.VMEM((1, k), jnp.int32),
            pltpu.VMEM((1, k), jnp.int32),
            pltpu.VMEM((_NB, k, h), jnp.float32),
            pltpu.VMEM_SHARED((n, h), jnp.float32),
            [pltpu.SemaphoreType.DMA] * _NB,
            [pltpu.SemaphoreType.DMA] * _NB,
        ],
    )
    def agg(y_hbm, ei_hbm, out_hbm, sidx, didx, sidx_x, didx_x, rows, accum,
            gsems, ssems):
        cid = lax.axis_index("c")
        sid = lax.axis_index("s")
        wid = sid * _NC + cid
        rowbase = wid * nfull
        pltpu.sync_copy(ei_hbm.at[0, pl.ds(rowbase, nfull)], sidx)
        pltpu.sync_copy(ei_hbm.at[1, pl.ds(rowbase, nfull)], didx)

        @pl.when(wid < nextra)
        def _():
            pltpu.sync_copy(ei_hbm.at[0, pl.ds(_NW * nfull + wid, 1)], sidx_x)
            pltpu.sync_copy(ei_hbm.at[1, pl.ds(_NW * nfull + wid, 1)], didx_x)

        def zbody(r, carry):
            for c4 in range(h // 16):
                rows[0, r, pl.ds(c4 * 16, 16)] = jnp.zeros((16,), jnp.float32)
            return carry
        lax.fori_loop(0, k, zbody, 0)
        base = sid * rows_per_tile
        for zi in range(zfull):
            pltpu.sync_copy(rows.at[0], accum.at[pl.ds(base + zi * k, k)])
        if zrem:
            pltpu.sync_copy(rows.at[0, pl.ds(0, zrem)],
                            accum.at[pl.ds(base + zfull * k, zrem)])
        plsc.subcore_barrier()

        for b in range(_AHEAD):
            pltpu.async_copy(y_hbm.at[sidx.at[b]], rows.at[b], gsems[b])

        def body(t, carry):
            for b in range(_NB):
                j = t * _NB + b
                pltpu.make_async_copy(y_hbm.at[sidx.at[j]], rows.at[b],
                                      gsems[b]).wait()
                pltpu.async_copy(rows.at[b], accum.at[didx.at[j]], ssems[b],
                                 add=True)
                f = j + _AHEAD
                bf = (b + _AHEAD) % _NB

                @pl.when(f < nfull)
                def _():
                    @pl.when(j >= _NB - _AHEAD)
                    def _():
                        pltpu.make_async_copy(
                            rows.at[bf], accum.at[didx.at[j - (_NB - _AHEAD)]],
                            ssems[bf]).wait()
                    pltpu.async_copy(y_hbm.at[sidx.at[f]], rows.at[bf],
                                     gsems[bf])
            return carry
        lax.fori_loop(0, nfull // _NB, body, 0)
        for b in range(_NB):
            pltpu.make_async_copy(rows.at[b],
                                  accum.at[didx.at[nfull - _NB + b]],
                                  ssems[b]).wait()

        @pl.when(wid < nextra)
        def _():
            pltpu.async_copy(y_hbm.at[sidx_x.at[0]], rows.at[0],
                             gsems[0]).wait()
            pltpu.sync_copy(rows.at[0], accum.at[didx_x.at[0]], add=True)
        plsc.subcore_barrier()

        wchunk = (n // _NS) & ~7
        wbase = sid * wchunk
        pltpu.sync_copy(accum.at[pl.ds(wbase, wchunk)],
                        out_hbm.at[cid, pl.ds(wbase, wchunk)])
        rem = n - _NS * wchunk
        if rem:
            @pl.when(sid == _NS - 1)
            def _():
                pltpu.sync_copy(accum.at[pl.ds(_NS * wchunk, rem)],
                                out_hbm.at[cid, pl.ds(_NS * wchunk, rem)])

    return agg


def _blockdiag2(w):
    zw = jnp.zeros_like(w)
    return jnp.concatenate(
        [jnp.concatenate([w, zw], axis=1),
         jnp.concatenate([zw, w], axis=1)], axis=0)


def kernel(x, edge_index, batch, W1a, b1a, W1b, b1b, W2a, b2a, W2b, b2b,
           W3a, b3a, W3b, b3b, Wl, bl):
    n, d = x.shape
    h = W1a.shape[1]
    e = edge_index.shape[1]
    n2 = n // 2

    k = 128
    c_total = e // k

    ei3 = edge_index.reshape(2, c_total, k)
    x256 = x.reshape(n2, 2 * d)
    batch4 = jnp.stack([batch[0::2], batch[1::2]], 0).reshape(
        2, n2 // _BN2, 1, _BN2)

    w1d = _blockdiag2(W1a)
    w1bd, w2ad, w2bd, w3ad, w3bd = map(
        _blockdiag2, (W1b, W2a, W2b, W3a, W3b))
    dup = lambda b: jnp.concatenate([b, b]).reshape(1, 2 * h)
    b1a2, b1b2, b2a2, b2b2, b3a2, b3b2 = map(
        dup, (b1a, b1b, b2a, b2b, b3a, b3b))
    bl2 = bl.reshape(1, -1)

    agg = _make_aggregate(n, h, c_total, k)

    def agg128(y128):
        p = agg(y128.reshape(n, h), ei3)
        return p.reshape(_NC, n2, 128)

    y1 = _first_matmul(x256, w1d)
    p1 = agg128(y1)
    y2 = _fuse(y1, p1, b1a2, w1bd, b1b2, w2ad)
    p2 = agg128(y2)
    y3 = _fuse(y2, p2, b2a2, w2bd, b2b2, w3ad)
    p3 = agg128(y3)
    return _final(y3, p3, b3a2, w3bd, b3b2, batch4, Wl, bl2)


# speedup vs baseline: 1.0290x; 1.0290x over previous
import functools

import jax
import jax.numpy as jnp
from jax import lax
from jax.experimental import pallas as pl
from jax.experimental.pallas import tpu as pltpu
from jax.experimental.pallas import tpu_sc as plsc

_NC = 2
_NS = 16
_NW = _NC * _NS

_BN2 = 1000
_G = 64


def _mm_body(x_ref, w_ref, o_ref):
    o_ref[...] = jnp.dot(x_ref[...], w_ref[...],
                         preferred_element_type=jnp.float32)


def _first_matmul(x256, w1d):
    n2, d2 = x256.shape
    return pl.pallas_call(
        _mm_body,
        grid=(n2 // _BN2,),
        in_specs=[pl.BlockSpec((_BN2, d2), lambda i: (i, 0)),
                  pl.BlockSpec((d2, 128), lambda i: (0, 0))],
        out_specs=pl.BlockSpec((_BN2, 128), lambda i: (i, 0)),
        out_shape=jax.ShapeDtypeStruct((n2, 128), jnp.float32),
    )(x256, w1d)


def _fuse_body(y_ref, p_ref, ba_ref, wbd_ref, bb_ref, wnd_ref, o_ref):
    h = jnp.maximum(y_ref[...] + p_ref[0] + p_ref[1] + ba_ref[...], 0.0)
    t = jnp.dot(h, wbd_ref[...], preferred_element_type=jnp.float32)
    t = jnp.maximum(t + bb_ref[...], 0.0)
    o_ref[...] = jnp.dot(t, wnd_ref[...], preferred_element_type=jnp.float32)


def _fuse(y, p, ba2, wbd, bb2, wnd):
    n2 = y.shape[0]
    return pl.pallas_call(
        _fuse_body,
        grid=(n2 // _BN2,),
        in_specs=[pl.BlockSpec((_BN2, 128), lambda i: (i, 0)),
                  pl.BlockSpec((_NC, _BN2, 128), lambda i: (0, i, 0)),
                  pl.BlockSpec((1, 128), lambda i: (0, 0)),
                  pl.BlockSpec((128, 128), lambda i: (0, 0)),
                  pl.BlockSpec((1, 128), lambda i: (0, 0)),
                  pl.BlockSpec((128, 128), lambda i: (0, 0))],
        out_specs=pl.BlockSpec((_BN2, 128), lambda i: (i, 0)),
        out_shape=jax.ShapeDtypeStruct((n2, 128), jnp.float32),
    )(y, p, ba2, wbd, bb2, wnd)


def _final_body(y_ref, p_ref, ba_ref, wbd_ref, bb_ref, b_ref, wl_ref,
                bl_ref, o_ref, acc_ref):
    i = pl.program_id(0)

    @pl.when(i == 0)
    def _():
        acc_ref[...] = jnp.zeros_like(acc_ref)

    h = jnp.maximum(y_ref[...] + p_ref[0] + p_ref[1] + ba_ref[...], 0.0)
    z = jnp.dot(h, wbd_ref[...], preferred_element_type=jnp.float32)
    z = jnp.maximum(z + bb_ref[...], 0.0)
    ids_e = b_ref[0, 0]
    ids_o = b_ref[1, 0]
    seg = lax.broadcasted_iota(jnp.int32, (_G, _BN2), 0)
    oh_e = (seg == ids_e).astype(jnp.float32)
    oh_o = (seg == ids_o).astype(jnp.float32)
    acc_ref[...] += (
        jnp.dot(oh_e, z[:, :64], preferred_element_type=jnp.float32)
        + jnp.dot(oh_o, z[:, 64:], preferred_element_type=jnp.float32))

    @pl.when(i == pl.num_programs(0) - 1)
    def _():
        o_ref[...] = jnp.dot(acc_ref[...], wl_ref[...],
                             preferred_element_type=jnp.float32) + bl_ref[...]


def _final(y, p, ba2, wbd, bb2, batch4, wl, bl):
    n2 = y.shape[0]
    h = wl.shape[0]
    dout = wl.shape[1]
    return pl.pallas_call(
        _final_body,
        grid=(n2 // _BN2,),
        in_specs=[pl.BlockSpec((_BN2, 128), lambda i: (i, 0)),
                  pl.BlockSpec((_NC, _BN2, 128), lambda i: (0, i, 0)),
                  pl.BlockSpec((1, 128), lambda i: (0, 0)),
                  pl.BlockSpec((128, 128), lambda i: (0, 0)),
                  pl.BlockSpec((1, 128), lambda i: (0, 0)),
                  pl.BlockSpec((2, 1, 1, _BN2), lambda i: (0, i, 0, 0)),
                  pl.BlockSpec((h, dout), lambda i: (0, 0)),
                  pl.BlockSpec((1, dout), lambda i: (0, 0))],
        out_specs=pl.BlockSpec((_G, dout), lambda i: (0, 0)),
        out_shape=jax.ShapeDtypeStruct((_G, dout), jnp.float32),
        scratch_shapes=[pltpu.VMEM((_G, h), jnp.float32)],
    )(y, p, ba2, wbd, bb2, batch4, wl, bl)


_NB = 6
_AHEAD = 4


def _make_aggregate(n, h, c_total, k):
    rows_per_tile = n // _NS
    nfull = c_total // _NW
    nextra = c_total - nfull * _NW
    zfull = rows_per_tile // k
    zrem = rows_per_tile - zfull * k
    mesh = plsc.VectorSubcoreMesh(core_axis_name="c", subcore_axis_name="s")
    assert nfull % _NB == 0 and nextra <= _NW

    @functools.partial(
        pl.kernel,
        out_type=jax.ShapeDtypeStruct((_NC, n, h), jnp.float32),
        mesh=mesh,
        compiler_params=pltpu.CompilerParams(use_tc_tiling_on_sc=False),
        scratch_types=[
            pltpu.VMEM((nfull, k), jnp.int32),
            pltpu.VMEM((nfull, k), jnp.int32),
            pltpu.VMEM((1, k), jnp.int32),
            pltpu.VMEM((1, k), jnp.int32),
            pltpu.VMEM((_NB, k, h), jnp.float32),
            pltpu.VMEM((k, h), jnp.float32),
            pltpu.VMEM_SHARED((n, h), jnp.float32),
            [pltpu.SemaphoreType.DMA] * _NB,
            [pltpu.SemaphoreType.DMA] * _NB,
        ],
    )
    def agg(y_hbm, ei_hbm, out_hbm, sidx, didx, sidx_x, didx_x, rows, zbuf,
            accum, gsems, ssems):
        cid = lax.axis_index("c")
        sid = lax.axis_index("s")
        wid = sid * _NC + cid
        rowbase = wid * nfull
        pltpu.sync_copy(ei_hbm.at[0, pl.ds(rowbase, nfull)], sidx)
        pltpu.sync_copy(ei_hbm.at[1, pl.ds(rowbase, nfull)], didx)

        @pl.when(wid < nextra)
        def _():
            pltpu.sync_copy(ei_hbm.at[0, pl.ds(_NW * nfull + wid, 1)], sidx_x)
            pltpu.sync_copy(ei_hbm.at[1, pl.ds(_NW * nfull + wid, 1)], didx_x)

        for b in range(_AHEAD):
            pltpu.async_copy(y_hbm.at[sidx.at[b]], rows.at[b], gsems[b])

        def zbody(r, carry):
            for c4 in range(h // 16):
                zbuf[r, pl.ds(c4 * 16, 16)] = jnp.zeros((16,), jnp.float32)
            return carry
        lax.fori_loop(0, k, zbody, 0)
        base = sid * rows_per_tile
        for zi in range(zfull):
            pltpu.sync_copy(zbuf, accum.at[pl.ds(base + zi * k, k)])
        if zrem:
            pltpu.sync_copy(zbuf.at[pl.ds(0, zrem)],
                            accum.at[pl.ds(base + zfull * k, zrem)])
        plsc.subcore_barrier()


        def body(t, carry):
            for b in range(_NB):
                j = t * _NB + b
                pltpu.make_async_copy(y_hbm.at[sidx.at[j]], rows.at[b],
                                      gsems[b]).wait()
                pltpu.async_copy(rows.at[b], accum.at[didx.at[j]], ssems[b],
                                 add=True)
                f = j + _AHEAD
                bf = (b + _AHEAD) % _NB

                @pl.when(f < nfull)
                def _():
                    @pl.when(j >= _NB - _AHEAD)
                    def _():
                        pltpu.make_async_copy(
                            rows.at[bf], accum.at[didx.at[j - (_NB - _AHEAD)]],
                            ssems[bf]).wait()
                    pltpu.async_copy(y_hbm.at[sidx.at[f]], rows.at[bf],
                                     gsems[bf])
            return carry
        lax.fori_loop(0, nfull // _NB, body, 0)
        for b in range(_NB):
            pltpu.make_async_copy(rows.at[b],
                                  accum.at[didx.at[nfull - _NB + b]],
                                  ssems[b]).wait()

        @pl.when(wid < nextra)
        def _():
            pltpu.async_copy(y_hbm.at[sidx_x.at[0]], rows.at[0],
                             gsems[0]).wait()
            pltpu.sync_copy(rows.at[0], accum.at[didx_x.at[0]], add=True)
        plsc.subcore_barrier()

        wchunk = (n // _NS) & ~7
        wbase = sid * wchunk
        pltpu.sync_copy(accum.at[pl.ds(wbase, wchunk)],
                        out_hbm.at[cid, pl.ds(wbase, wchunk)])
        rem = n - _NS * wchunk
        if rem:
            @pl.when(sid == _NS - 1)
            def _():
                pltpu.sync_copy(accum.at[pl.ds(_NS * wchunk, rem)],
                                out_hbm.at[cid, pl.ds(_NS * wchunk, rem)])

    return agg


def _blockdiag2(w):
    zw = jnp.zeros_like(w)
    return jnp.concatenate(
        [jnp.concatenate([w, zw], axis=1),
         jnp.concatenate([zw, w], axis=1)], axis=0)


def kernel(x, edge_index, batch, W1a, b1a, W1b, b1b, W2a, b2a, W2b, b2b,
           W3a, b3a, W3b, b3b, Wl, bl):
    n, d = x.shape
    h = W1a.shape[1]
    e = edge_index.shape[1]
    n2 = n // 2

    k = 128
    c_total = e // k

    ei3 = edge_index.reshape(2, c_total, k)
    x256 = x.reshape(n2, 2 * d)
    batch4 = jnp.stack([batch[0::2], batch[1::2]], 0).reshape(
        2, n2 // _BN2, 1, _BN2)

    w1d = _blockdiag2(W1a)
    w1bd, w2ad, w2bd, w3ad, w3bd = map(
        _blockdiag2, (W1b, W2a, W2b, W3a, W3b))
    dup = lambda b: jnp.concatenate([b, b]).reshape(1, 2 * h)
    b1a2, b1b2, b2a2, b2b2, b3a2, b3b2 = map(
        dup, (b1a, b1b, b2a, b2b, b3a, b3b))
    bl2 = bl.reshape(1, -1)

    agg = _make_aggregate(n, h, c_total, k)

    def agg128(y128):
        p = agg(y128.reshape(n, h), ei3)
        return p.reshape(_NC, n2, 128)

    y1 = _first_matmul(x256, w1d)
    p1 = agg128(y1)
    y2 = _fuse(y1, p1, b1a2, w1bd, b1b2, w2ad)
    p2 = agg128(y2)
    y3 = _fuse(y2, p2, b2a2, w2bd, b2b2, w3ad)
    p3 = agg128(y3)
    return _final(y3, p3, b3a2, w3bd, b3b2, batch4, Wl, bl2)


# speedup vs baseline: 1.0391x; 1.0098x over previous
import functools

import jax
import jax.numpy as jnp
from jax import lax
from jax.experimental import pallas as pl
from jax.experimental.pallas import tpu as pltpu
from jax.experimental.pallas import tpu_sc as plsc

_NC = 2
_NS = 16
_NW = _NC * _NS

_BN2 = 1000
_G = 64


def _mm_body(x_ref, w_ref, o_ref):
    o_ref[...] = jnp.dot(x_ref[...], w_ref[...],
                         preferred_element_type=jnp.float32)


def _first_matmul(x256, w1d):
    n2, d2 = x256.shape
    return pl.pallas_call(
        _mm_body,
        grid=(n2 // _BN2,),
        in_specs=[pl.BlockSpec((_BN2, d2), lambda i: (i, 0)),
                  pl.BlockSpec((d2, 128), lambda i: (0, 0))],
        out_specs=pl.BlockSpec((_BN2, 128), lambda i: (i, 0)),
        out_shape=jax.ShapeDtypeStruct((n2, 128), jnp.float32),
    )(x256, w1d)


def _fuse_body(y_ref, p_ref, ba_ref, wbd_ref, bb_ref, wnd_ref, o_ref):
    h = jnp.maximum(y_ref[...] + p_ref[0] + p_ref[1] + ba_ref[...], 0.0)
    t = jnp.dot(h, wbd_ref[...], preferred_element_type=jnp.float32)
    t = jnp.maximum(t + bb_ref[...], 0.0)
    o_ref[...] = jnp.dot(t, wnd_ref[...], preferred_element_type=jnp.float32)


def _fuse(y, p, ba2, wbd, bb2, wnd):
    n2 = y.shape[0]
    return pl.pallas_call(
        _fuse_body,
        grid=(n2 // _BN2,),
        in_specs=[pl.BlockSpec((_BN2, 128), lambda i: (i, 0)),
                  pl.BlockSpec((_NC, _BN2, 128), lambda i: (0, i, 0)),
                  pl.BlockSpec((1, 128), lambda i: (0, 0)),
                  pl.BlockSpec((128, 128), lambda i: (0, 0)),
                  pl.BlockSpec((1, 128), lambda i: (0, 0)),
                  pl.BlockSpec((128, 128), lambda i: (0, 0))],
        out_specs=pl.BlockSpec((_BN2, 128), lambda i: (i, 0)),
        out_shape=jax.ShapeDtypeStruct((n2, 128), jnp.float32),
    )(y, p, ba2, wbd, bb2, wnd)


def _final_body(y_ref, p_ref, ba_ref, wbd_ref, bb_ref, b_ref, wl_ref,
                bl_ref, o_ref, acc_ref):
    i = pl.program_id(0)

    @pl.when(i == 0)
    def _():
        acc_ref[...] = jnp.zeros_like(acc_ref)

    h = jnp.maximum(y_ref[...] + p_ref[0] + p_ref[1] + ba_ref[...], 0.0)
    z = jnp.dot(h, wbd_ref[...], preferred_element_type=jnp.float32)
    z = jnp.maximum(z + bb_ref[...], 0.0)
    ids_e = b_ref[0, 0]
    ids_o = b_ref[1, 0]
    seg = lax.broadcasted_iota(jnp.int32, (_G, _BN2), 0)
    oh_e = (seg == ids_e).astype(jnp.float32)
    oh_o = (seg == ids_o).astype(jnp.float32)
    acc_ref[...] += (
        jnp.dot(oh_e, z[:, :64], preferred_element_type=jnp.float32)
        + jnp.dot(oh_o, z[:, 64:], preferred_element_type=jnp.float32))

    @pl.when(i == pl.num_programs(0) - 1)
    def _():
        o_ref[...] = jnp.dot(acc_ref[...], wl_ref[...],
                             preferred_element_type=jnp.float32) + bl_ref[...]


def _final(y, p, ba2, wbd, bb2, batch4, wl, bl):
    n2 = y.shape[0]
    h = wl.shape[0]
    dout = wl.shape[1]
    return pl.pallas_call(
        _final_body,
        grid=(n2 // _BN2,),
        in_specs=[pl.BlockSpec((_BN2, 128), lambda i: (i, 0)),
                  pl.BlockSpec((_NC, _BN2, 128), lambda i: (0, i, 0)),
                  pl.BlockSpec((1, 128), lambda i: (0, 0)),
                  pl.BlockSpec((128, 128), lambda i: (0, 0)),
                  pl.BlockSpec((1, 128), lambda i: (0, 0)),
                  pl.BlockSpec((2, 1, 1, _BN2), lambda i: (0, i, 0, 0)),
                  pl.BlockSpec((h, dout), lambda i: (0, 0)),
                  pl.BlockSpec((1, dout), lambda i: (0, 0))],
        out_specs=pl.BlockSpec((_G, dout), lambda i: (0, 0)),
        out_shape=jax.ShapeDtypeStruct((_G, dout), jnp.float32),
        scratch_shapes=[pltpu.VMEM((_G, h), jnp.float32)],
    )(y, p, ba2, wbd, bb2, batch4, wl, bl)


_NB = 6
_AHEAD = 4


def _make_aggregate(n, h, c_total, k):
    rows_per_tile = n // _NS
    nfull = c_total // _NW
    nextra = c_total - nfull * _NW
    zfull = rows_per_tile // k
    zrem = rows_per_tile - zfull * k
    mesh = plsc.VectorSubcoreMesh(core_axis_name="c", subcore_axis_name="s")
    assert nfull % _NB == 0 and nextra <= _NW

    @functools.partial(
        pl.kernel,
        out_type=jax.ShapeDtypeStruct((_NC, n, h), jnp.float32),
        mesh=mesh,
        compiler_params=pltpu.CompilerParams(use_tc_tiling_on_sc=False),
        scratch_types=[
            pltpu.VMEM((nfull, k), jnp.int32),
            pltpu.VMEM((nfull, k), jnp.int32),
            pltpu.VMEM((1, k), jnp.int32),
            pltpu.VMEM((1, k), jnp.int32),
            pltpu.VMEM((_NB, k, h), jnp.float32),
            pltpu.VMEM((k, h), jnp.float32),
            pltpu.VMEM((k, h), jnp.float32),
            pltpu.VMEM_SHARED((n, h), jnp.float32),
            [pltpu.SemaphoreType.DMA] * _NB,
            [pltpu.SemaphoreType.DMA] * _NB,
            pltpu.SemaphoreType.DMA,
        ],
    )
    def agg(y_hbm, ei_hbm, out_hbm, sidx, didx, sidx_x, didx_x, rows, zbuf,
            lrows, accum, gsems, ssems, lsem):
        cid = lax.axis_index("c")
        sid = lax.axis_index("s")
        wid = sid * _NC + cid
        rowbase = wid * nfull
        pltpu.sync_copy(ei_hbm.at[0, pl.ds(rowbase, nfull)], sidx)
        pltpu.sync_copy(ei_hbm.at[1, pl.ds(rowbase, nfull)], didx)

        @pl.when(wid < nextra)
        def _():
            pltpu.sync_copy(ei_hbm.at[0, pl.ds(_NW * nfull + wid, 1)], sidx_x)
            pltpu.sync_copy(ei_hbm.at[1, pl.ds(_NW * nfull + wid, 1)], didx_x)

        for b in range(_AHEAD):
            pltpu.async_copy(y_hbm.at[sidx.at[b]], rows.at[b], gsems[b])

        @pl.when(wid < nextra)
        def _():
            pltpu.async_copy(y_hbm.at[sidx_x.at[0]], lrows, lsem)

        def zbody(r, carry):
            for c4 in range(h // 16):
                zbuf[r, pl.ds(c4 * 16, 16)] = jnp.zeros((16,), jnp.float32)
            return carry
        lax.fori_loop(0, k, zbody, 0)
        base = sid * rows_per_tile
        for zi in range(zfull):
            pltpu.sync_copy(zbuf, accum.at[pl.ds(base + zi * k, k)])
        if zrem:
            pltpu.sync_copy(zbuf.at[pl.ds(0, zrem)],
                            accum.at[pl.ds(base + zfull * k, zrem)])
        plsc.subcore_barrier()


        def body(t, carry):
            for b in range(_NB):
                j = t * _NB + b
                pltpu.make_async_copy(y_hbm.at[sidx.at[j]], rows.at[b],
                                      gsems[b]).wait()
                pltpu.async_copy(rows.at[b], accum.at[didx.at[j]], ssems[b],
                                 add=True)
                f = j + _AHEAD
                bf = (b + _AHEAD) % _NB

                @pl.when(f < nfull)
                def _():
                    @pl.when(j >= _NB - _AHEAD)
                    def _():
                        pltpu.make_async_copy(
                            rows.at[bf], accum.at[didx.at[j - (_NB - _AHEAD)]],
                            ssems[bf]).wait()
                    pltpu.async_copy(y_hbm.at[sidx.at[f]], rows.at[bf],
                                     gsems[bf])
            return carry
        lax.fori_loop(0, nfull // _NB, body, 0)
        for b in range(_NB):
            pltpu.make_async_copy(rows.at[b],
                                  accum.at[didx.at[nfull - _NB + b]],
                                  ssems[b]).wait()

        @pl.when(wid < nextra)
        def _():
            pltpu.make_async_copy(y_hbm.at[sidx_x.at[0]], lrows, lsem).wait()
            pltpu.sync_copy(lrows, accum.at[didx_x.at[0]], add=True)
        plsc.subcore_barrier()

        wchunk = (n // _NS) & ~7
        wbase = sid * wchunk
        pltpu.sync_copy(accum.at[pl.ds(wbase, wchunk)],
                        out_hbm.at[cid, pl.ds(wbase, wchunk)])
        rem = n - _NS * wchunk
        if rem:
            @pl.when(sid == _NS - 1)
            def _():
                pltpu.sync_copy(accum.at[pl.ds(_NS * wchunk, rem)],
                                out_hbm.at[cid, pl.ds(_NS * wchunk, rem)])

    return agg


def _blockdiag2(w):
    zw = jnp.zeros_like(w)
    return jnp.concatenate(
        [jnp.concatenate([w, zw], axis=1),
         jnp.concatenate([zw, w], axis=1)], axis=0)


def kernel(x, edge_index, batch, W1a, b1a, W1b, b1b, W2a, b2a, W2b, b2b,
           W3a, b3a, W3b, b3b, Wl, bl):
    n, d = x.shape
    h = W1a.shape[1]
    e = edge_index.shape[1]
    n2 = n // 2

    k = 128
    c_total = e // k

    ei3 = edge_index.reshape(2, c_total, k)
    x256 = x.reshape(n2, 2 * d)
    batch4 = jnp.stack([batch[0::2], batch[1::2]], 0).reshape(
        2, n2 // _BN2, 1, _BN2)

    w1d = _blockdiag2(W1a)
    w1bd, w2ad, w2bd, w3ad, w3bd = map(
        _blockdiag2, (W1b, W2a, W2b, W3a, W3b))
    dup = lambda b: jnp.concatenate([b, b]).reshape(1, 2 * h)
    b1a2, b1b2, b2a2, b2b2, b3a2, b3b2 = map(
        dup, (b1a, b1b, b2a, b2b, b3a, b3b))
    bl2 = bl.reshape(1, -1)

    agg = _make_aggregate(n, h, c_total, k)

    def agg128(y128):
        p = agg(y128.reshape(n, h), ei3)
        return p.reshape(_NC, n2, 128)

    y1 = _first_matmul(x256, w1d)
    p1 = agg128(y1)
    y2 = _fuse(y1, p1, b1a2, w1bd, b1b2, w2ad)
    p2 = agg128(y2)
    y3 = _fuse(y2, p2, b2a2, w2bd, b2b2, w3ad)
    p3 = agg128(y3)
    return _final(y3, p3, b3a2, w3bd, b3b2, batch4, Wl, bl2)
